# Initial kernel scaffold; baseline (speedup 1.0000x reference)
import jax
import jax.numpy as jnp
from jax.experimental import pallas as pl


def kernel(x, E_relative_position):
    raise NotImplementedError("write your pallas kernel here")


# speedup vs baseline: 4.2465x; 4.2465x over previous
import functools

import jax
import jax.numpy as jnp
from jax import lax
from jax.experimental import pallas as pl
from jax.experimental.pallas import tpu as pltpu
from jax.experimental.pallas import tpu_sc as plsc

ROWS = 4096
DIM = 64
B = 4096 * 200
NW = 32
PER_W = B // NW
IW = 128
K = 8
CHUNK = K * IW
NITER = PER_W // CHUNK
XROWS = B // IW


def _make_sc_gather():
    mesh = plsc.VectorSubcoreMesh(core_axis_name="c", subcore_axis_name="s")

    @functools.partial(
        pl.kernel,
        mesh=mesh,
        out_type=jax.ShapeDtypeStruct((B, DIM), jnp.float32),
        scratch_types=[
            pltpu.VMEM((K, IW), jnp.int32),
            pltpu.VMEM((CHUNK, DIM), jnp.float32),
            pltpu.SemaphoreType.DMA,
        ],
        compiler_params=pltpu.CompilerParams(use_tc_tiling_on_sc=False),
    )
    def gather_kernel(x_hbm, table_hbm, out_hbm, idx_v, rows_v, sem):
        wid = lax.axis_index("s") * 2 + lax.axis_index("c")
        base_row = wid * (PER_W // IW)

        def body(i, carry):
            row = base_row + i * K
            pltpu.sync_copy(x_hbm.at[pl.ds(row, K), :], idx_v)
            copies = []
            for j in range(K):
                copies.append(
                    pltpu.async_copy(
                        table_hbm.at[idx_v.at[j]],
                        rows_v.at[pl.ds(j * IW, IW), :],
                        sem,
                    )
                )
            for c in copies:
                c.wait()
            pltpu.sync_copy(rows_v, out_hbm.at[pl.ds(row * IW, CHUNK), :])
            return carry

        lax.fori_loop(0, NITER, body, 0)

    return gather_kernel


_sc_gather = _make_sc_gather()


@jax.jit
def kernel(x, E_relative_position):
    n, m = x.shape
    x_flat = x.reshape(XROWS, IW).astype(jnp.int32)
    out = _sc_gather(x_flat, E_relative_position)
    return out.reshape(n, m, DIM)


# speedup vs baseline: 4.3493x; 1.0242x over previous
import functools

import jax
import jax.numpy as jnp
from jax import lax
from jax.experimental import pallas as pl
from jax.experimental.pallas import tpu as pltpu
from jax.experimental.pallas import tpu_sc as plsc

ROWS = 4096
DIM = 64
B = 4096 * 200
NW = 32
PER_W = B // NW
IW = 128
K = 4
CHUNK = K * IW
NITER = PER_W // CHUNK
XROWS = B // IW


def _make_sc_gather():
    mesh = plsc.VectorSubcoreMesh(core_axis_name="c", subcore_axis_name="s")

    @functools.partial(
        pl.kernel,
        mesh=mesh,
        out_type=jax.ShapeDtypeStruct((B, DIM), jnp.float32),
        scratch_types=[
            pltpu.VMEM((2, K, IW), jnp.int32),
            pltpu.VMEM((2, CHUNK, DIM), jnp.float32),
            pltpu.SemaphoreType.DMA,
            pltpu.SemaphoreType.DMA,
        ],
        compiler_params=pltpu.CompilerParams(use_tc_tiling_on_sc=False),
    )
    def gather_kernel(x_hbm, table_hbm, out_hbm, idx_v, rows_v, sem0, sem1):
        wid = lax.axis_index("s") * 2 + lax.axis_index("c")
        base_row = wid * (PER_W // IW)
        sems = (sem0, sem1)

        def start_chunk(i, b):
            row = base_row + i * K
            pltpu.sync_copy(x_hbm.at[pl.ds(row, K), :], idx_v.at[b])
            copies = []
            for j in range(K):
                copies.append(
                    pltpu.async_copy(
                        table_hbm.at[idx_v.at[b, j]],
                        rows_v.at[b, pl.ds(j * IW, IW), :],
                        sems[b],
                    )
                )
            return copies

        def wait_chunk(i, b):
            row = base_row + i * K
            for j in range(K):
                pltpu.make_async_copy(
                    table_hbm.at[idx_v.at[b, j]],
                    rows_v.at[b, pl.ds(j * IW, IW), :],
                    sems[b],
                ).wait()

        def write_chunk(i, b):
            row = base_row + i * K
            pltpu.sync_copy(rows_v.at[b], out_hbm.at[pl.ds(row * IW, CHUNK), :])

        start_chunk(0, 0)

        def body(ii, carry):
            for b in (0, 1):
                i = 2 * ii + b
                nb = 1 - b

                @pl.when(i + 1 < NITER)
                def _():
                    start_chunk(i + 1, nb)

                wait_chunk(i, b)
                write_chunk(i, b)
            return carry

        lax.fori_loop(0, (NITER + 1) // 2, body, 0)

    return gather_kernel


_sc_gather = _make_sc_gather()


@jax.jit
def kernel(x, E_relative_position):
    n, m = x.shape
    x_flat = x.reshape(XROWS, IW).astype(jnp.int32)
    out = _sc_gather(x_flat, E_relative_position)
    return out.reshape(n, m, DIM)


# speedup vs baseline: 4.3643x; 1.0034x over previous
import functools

import jax
import jax.numpy as jnp
from jax import lax
from jax.experimental import pallas as pl
from jax.experimental.pallas import tpu as pltpu
from jax.experimental.pallas import tpu_sc as plsc

N = 4096
M = 200
DIM = 64
NW = 32
RPW = N // NW
RPC = 4
CHUNK = RPC * M
NITER = RPW // RPC


def _make_sc_gather():
    mesh = plsc.VectorSubcoreMesh(core_axis_name="c", subcore_axis_name="s")

    @functools.partial(
        pl.kernel,
        mesh=mesh,
        out_type=jax.ShapeDtypeStruct((N, M, DIM), jnp.float32),
        scratch_types=[
            pltpu.VMEM((2, RPC, M), jnp.int32),
            pltpu.VMEM((2, RPC, M, DIM), jnp.float32),
            pltpu.SemaphoreType.DMA,
            pltpu.SemaphoreType.DMA,
        ],
        compiler_params=pltpu.CompilerParams(use_tc_tiling_on_sc=False),
    )
    def gather_kernel(x_hbm, table_hbm, out_hbm, idx_v, rows_v, sem0, sem1):
        wid = lax.axis_index("s") * 2 + lax.axis_index("c")
        base_row = wid * RPW
        sems = (sem0, sem1)

        def copies_for(b):
            out = []
            for k in range(RPC):
                out.append(
                    pltpu.make_async_copy(
                        table_hbm.at[idx_v.at[b, k, pl.ds(0, 128)]],
                        rows_v.at[b, k, pl.ds(0, 128), :],
                        sems[b],
                    )
                )
                out.append(
                    pltpu.make_async_copy(
                        table_hbm.at[idx_v.at[b, k, pl.ds(128, M - 128)]],
                        rows_v.at[b, k, pl.ds(128, M - 128), :],
                        sems[b],
                    )
                )
            return out

        def start_chunk(i, b):
            row = base_row + i * RPC
            pltpu.sync_copy(x_hbm.at[pl.ds(row, RPC), :], idx_v.at[b])
            for c in copies_for(b):
                c.start()

        def finish_chunk(i, b):
            for c in copies_for(b):
                c.wait()
            row = base_row + i * RPC
            pltpu.sync_copy(rows_v.at[b], out_hbm.at[pl.ds(row, RPC)])

        start_chunk(0, 0)

        def body(ii, carry):
            for b in (0, 1):
                i = 2 * ii + b

                @pl.when(i + 1 < NITER)
                def _():
                    start_chunk(i + 1, 1 - b)

                finish_chunk(i, b)
            return carry

        lax.fori_loop(0, (NITER + 1) // 2, body, 0)

    return gather_kernel


_sc_gather = _make_sc_gather()


@jax.jit
def kernel(x, E_relative_position):
    return _sc_gather(x.astype(jnp.int32), E_relative_position)
